# Initial kernel scaffold; baseline (speedup 1.0000x reference)
import jax
import jax.numpy as jnp
from jax.experimental import pallas as pl


def kernel(x, edge_index, batch, params):
    raise NotImplementedError("write your pallas kernel here")


# speedup vs baseline: 3.0651x; 3.0651x over previous
import functools

import jax
import jax.numpy as jnp
from jax import lax
from jax.experimental import pallas as pl
from jax.experimental.pallas import tpu as pltpu
from jax.experimental.pallas import tpu_sc as plsc

N = 10000
NPAD = 10240
E = 320000
G = 64
SUB = 16
LN = 128
NCH = 160
EPAD = SUB * NCH * LN
ROWS_PER_SUB = NPAD // SUB
HPREC = lax.Precision.HIGHEST


def _sc_mesh():
    return plsc.VectorSubcoreMesh(core_axis_name="c", subcore_axis_name="s")


def _agg_body(tbl, comb, i0, i1, g0, g1, acc,
              semi0, semi1, semg0, semg1, base, npairs):
    limit = base + 2 * npairs
    pltpu.async_copy(comb.at[base], i0, semi0)
    pltpu.async_copy(comb.at[base + 1], i1, semi1)
    pltpu.make_async_copy(comb.at[base], i0, semi0).wait()
    pltpu.async_copy(tbl.at[i0.at[0]], g0, semg0)
    pltpu.make_async_copy(comb.at[base + 1], i1, semi1).wait()
    pltpu.async_copy(tbl.at[i1.at[0]], g1, semg1)

    @pl.loop(0, npairs)
    def _(p):
        j0 = base + 2 * p
        j1 = j0 + 1
        pltpu.make_async_copy(tbl.at[i0.at[0]], g0, semg0).wait()
        pltpu.sync_copy(g0, acc.at[i0.at[1]], add=True)

        @pl.when(j0 + 2 < limit)
        def _():
            pltpu.async_copy(comb.at[j0 + 2], i0, semi0)

        pltpu.make_async_copy(tbl.at[i1.at[0]], g1, semg1).wait()
        pltpu.sync_copy(g1, acc.at[i1.at[1]], add=True)

        @pl.when(j1 + 2 < limit)
        def _():
            pltpu.async_copy(comb.at[j1 + 2], i1, semi1)

        @pl.when(j0 + 2 < limit)
        def _():
            pltpu.make_async_copy(comb.at[j0 + 2], i0, semi0).wait()
            pltpu.async_copy(tbl.at[i0.at[0]], g0, semg0)

        @pl.when(j1 + 2 < limit)
        def _():
            pltpu.make_async_copy(comb.at[j1 + 2], i1, semi1).wait()
            pltpu.async_copy(tbl.at[i1.at[0]], g1, semg1)


_SC_SCRATCH = [
    pltpu.VMEM((2, LN), jnp.int32),
    pltpu.VMEM((2, LN), jnp.int32),
    pltpu.VMEM((LN, 128), jnp.float32),
    pltpu.VMEM((LN, 128), jnp.float32),
    pltpu.VMEM_SHARED((NPAD, 128), jnp.float32),
    pltpu.SemaphoreType.DMA,
    pltpu.SemaphoreType.DMA,
    pltpu.SemaphoreType.DMA,
    pltpu.SemaphoreType.DMA,
]


def _sc_agg_halves(table2, comb4, zrows):
    @functools.partial(
        pl.kernel,
        out_type=jax.ShapeDtypeStruct((2, NPAD, 128), jnp.float32),
        mesh=_sc_mesh(),
        scratch_types=list(_SC_SCRATCH),
    )
    def k(tbl_hbm, comb_hbm, z_hbm, out_hbm,
          i0, i1, g0, g1, acc, semi0, semi1, semg0, semg1):
        c = lax.axis_index("c")
        s = lax.axis_index("s")
        pltpu.sync_copy(z_hbm, acc.at[pl.ds(s * ROWS_PER_SUB, ROWS_PER_SUB)])
        plsc.subcore_barrier()
        _agg_body(tbl_hbm.at[c], comb_hbm.at[s], i0, i1, g0, g1, acc,
                  semi0, semi1, semg0, semg1, 0, NCH // 2)
        plsc.subcore_barrier()
        pltpu.sync_copy(acc.at[pl.ds(s * ROWS_PER_SUB, ROWS_PER_SUB)],
                        out_hbm.at[c].at[pl.ds(s * ROWS_PER_SUB, ROWS_PER_SUB)])

    return k(table2, comb4, zrows)


def _sc_agg_partials(table, comb4, zrows):
    @functools.partial(
        pl.kernel,
        out_type=jax.ShapeDtypeStruct((2, NPAD, 128), jnp.float32),
        mesh=_sc_mesh(),
        scratch_types=list(_SC_SCRATCH),
    )
    def k(tbl_hbm, comb_hbm, z_hbm, out_hbm,
          i0, i1, g0, g1, acc, semi0, semi1, semg0, semg1):
        c = lax.axis_index("c")
        s = lax.axis_index("s")
        pltpu.sync_copy(z_hbm, acc.at[pl.ds(s * ROWS_PER_SUB, ROWS_PER_SUB)])
        plsc.subcore_barrier()
        _agg_body(tbl_hbm, comb_hbm.at[s], i0, i1, g0, g1, acc,
                  semi0, semi1, semg0, semg1, c * (NCH // 2), NCH // 4)
        plsc.subcore_barrier()
        pltpu.sync_copy(acc.at[pl.ds(s * ROWS_PER_SUB, ROWS_PER_SUB)],
                        out_hbm.at[c].at[pl.ds(s * ROWS_PER_SUB, ROWS_PER_SUB)])

    return k(table, comb4, zrows)


def _tc_layer0(part, x, WrT, WtT, b):
    def body(p_ref, x_ref, wr_ref, wt_ref, b_ref, o_ref):
        agg = p_ref[0] + p_ref[1]
        out = (jnp.dot(agg, wr_ref[...]) + b_ref[...]
               + jnp.dot(x_ref[...], wt_ref[...]))
        out = jnp.maximum(out, 0.0)
        o_ref[0, :, :] = out[:, :128]
        o_ref[1, :, :] = out[:, 128:]

    return pl.pallas_call(
        body,
        grid=(NPAD // 256,),
        in_specs=[
            pl.BlockSpec((2, 256, 128), lambda i: (0, i, 0)),
            pl.BlockSpec((256, 128), lambda i: (i, 0)),
            pl.BlockSpec((128, 256), lambda i: (0, 0)),
            pl.BlockSpec((128, 256), lambda i: (0, 0)),
            pl.BlockSpec((1, 256), lambda i: (0, 0)),
        ],
        out_specs=pl.BlockSpec((2, 256, 128), lambda i: (0, i, 0)),
        out_shape=jax.ShapeDtypeStruct((2, NPAD, 128), jnp.float32),
    )(part, x, WrT, WtT, b)


def _tc_layer(agg, h, WrT, WtT, b, relu):
    def body(a_ref, h_ref, wr_ref, wt_ref, b_ref, o_ref):
        acat = jnp.concatenate([a_ref[0], a_ref[1]], axis=1)
        hcat = jnp.concatenate([h_ref[0], h_ref[1]], axis=1)
        out = (jnp.dot(acat, wr_ref[...]) + b_ref[...]
               + jnp.dot(hcat, wt_ref[...]))
        if relu:
            out = jnp.maximum(out, 0.0)
        o_ref[0, :, :] = out[:, :128]
        o_ref[1, :, :] = out[:, 128:]

    return pl.pallas_call(
        body,
        grid=(NPAD // 256,),
        in_specs=[
            pl.BlockSpec((2, 256, 128), lambda i: (0, i, 0)),
            pl.BlockSpec((2, 256, 128), lambda i: (0, i, 0)),
            pl.BlockSpec((256, 256), lambda i: (0, 0)),
            pl.BlockSpec((256, 256), lambda i: (0, 0)),
            pl.BlockSpec((1, 256), lambda i: (0, 0)),
        ],
        out_specs=pl.BlockSpec((2, 256, 128), lambda i: (0, i, 0)),
        out_shape=jax.ShapeDtypeStruct((2, NPAD, 128), jnp.float32),
    )(agg, h, WrT, WtT, b)


def _tc_pool(h, batch3, w_out, b_out):
    nblk = NPAD // 256

    def body(h_ref, b_ref, w_ref, bo_ref, o_ref, sums, counts):
        i = pl.program_id(0)

        @pl.when(i == 0)
        def _():
            sums[...] = jnp.zeros_like(sums)
            counts[...] = jnp.zeros_like(counts)

        hcat = jnp.concatenate([h_ref[0], h_ref[1]], axis=1)
        bt = b_ref[0, 0, :]
        gid = lax.broadcasted_iota(jnp.int32, (G, 256), 0)
        onehot = (gid == bt[None, :]).astype(jnp.float32)
        sums[...] += jnp.dot(onehot, hcat, precision=HPREC)
        counts[...] += jnp.sum(onehot, axis=1, keepdims=True)

        @pl.when(i == nblk - 1)
        def _():
            cnt = jnp.maximum(counts[:, 0:1], 1.0)
            pooled = sums[...] / cnt
            pb = pooled.astype(jnp.bfloat16).astype(jnp.float32)
            wb = w_ref[...].astype(jnp.bfloat16).astype(jnp.float32)
            val = jnp.sum(pb * wb, axis=1, keepdims=True)
            o_ref[...] = val + bo_ref[...]

    return pl.pallas_call(
        body,
        grid=(nblk,),
        in_specs=[
            pl.BlockSpec((2, 256, 128), lambda i: (0, i, 0)),
            pl.BlockSpec((1, 1, 256), lambda i: (i, 0, 0)),
            pl.BlockSpec((1, 256), lambda i: (0, 0)),
            pl.BlockSpec((1, 1), lambda i: (0, 0)),
        ],
        out_specs=pl.BlockSpec((G, 1), lambda i: (0, 0)),
        out_shape=jax.ShapeDtypeStruct((G, 1), jnp.float32),
        scratch_shapes=[
            pltpu.VMEM((G, 256), jnp.float32),
            pltpu.VMEM((G, 128), jnp.float32),
        ],
    )(h, batch3, w_out, b_out)


def kernel(x, edge_index, batch, params):
    layers = params["layers"]
    src = edge_index[0]
    dst = edge_index[1]
    src3 = jnp.concatenate(
        [src, jnp.zeros((EPAD - E,), jnp.int32)]).reshape(SUB, NCH, LN)
    dst3 = jnp.concatenate(
        [dst, jnp.full((EPAD - E,), NPAD - 1, jnp.int32)]).reshape(SUB, NCH, LN)
    comb4 = jnp.stack([src3, dst3], axis=2)
    zrows = jnp.zeros((ROWS_PER_SUB, 128), jnp.float32)
    x_pad = jnp.zeros((NPAD, 128), jnp.float32).at[:N].set(x)
    batch3 = jnp.concatenate(
        [batch, jnp.full((NPAD - N,), G, jnp.int32)]).reshape(NPAD // 256, 1, 256)

    W_rel, b_rel, W_root = layers[0]
    part = _sc_agg_partials(x_pad, comb4, zrows)
    h = _tc_layer0(part, x_pad, W_rel.T, W_root.T, b_rel.reshape(1, -1))

    for i in range(1, len(layers)):
        W_rel, b_rel, W_root = layers[i]
        agg = _sc_agg_halves(h, comb4, zrows)
        h = _tc_layer(agg, h, W_rel.T, W_root.T, b_rel.reshape(1, -1),
                      relu=(i < len(layers) - 1))

    out = _tc_pool(h, batch3, params["W_out"], params["b_out"].reshape(1, 1))
    return out.reshape(-1)
